# baseline (speedup 1.0000x reference)
import functools

import jax
import jax.numpy as jnp
from jax import lax
from jax.experimental import pallas as pl
from jax.experimental.pallas import tpu as pltpu
from jax.experimental.pallas import tpu_sc as plsc


def _gather_rows_sc(input_ids, emb_table):
    B = input_ids.shape[0]
    V, H = emb_table.shape
    info = plsc.get_sparse_core_info()
    nw = info.num_cores * info.num_subcores
    b_per_w = B // nw

    mesh = plsc.VectorSubcoreMesh(core_axis_name="c", subcore_axis_name="s")

    @functools.partial(
        pl.kernel,
        mesh=mesh,
        out_type=jax.ShapeDtypeStruct((B, H), jnp.float32),
        compiler_params=pltpu.CompilerParams(use_tc_tiling_on_sc=False),
        scratch_types=[
            pltpu.VMEM((b_per_w,), jnp.int32),
            pltpu.VMEM((b_per_w, H), jnp.float32),
            pltpu.SemaphoreType.DMA,
        ],
    )
    def gather_k(idx_hbm, table_hbm, out_hbm, idx_v, rows_v, sem):
        wid = lax.axis_index("s") * info.num_cores + lax.axis_index("c")
        base = wid * b_per_w
        pltpu.sync_copy(idx_hbm.at[pl.ds(base, b_per_w)], idx_v)
        pltpu.async_copy(table_hbm.at[idx_v], rows_v, sem).wait()
        pltpu.sync_copy(rows_v, out_hbm.at[pl.ds(base, b_per_w)])

    return gather_k(input_ids, emb_table)


def _project_main(x, W, b, block_v=2048, ring=5):
    B, H = x.shape
    V = W.shape[0]
    v_al = (V // 128) * 128
    nv = pl.cdiv(v_al, block_v)
    nv_full = v_al // block_v
    tail = v_al - nv_full * block_v

    def mm_k(x_ref, w_ref, b_ref, o_hbm, o_vmem, sems):
        i = pl.program_id(0)
        s = lax.rem(i, ring)

        @pl.when(i >= ring)
        def _wait_slot():
            pltpu.make_async_copy(
                o_vmem.at[s], o_hbm.at[:, pl.ds(0, block_v)], sems.at[s]
            ).wait()

        o_vmem[s, :, :] = (
            lax.dot_general(
                x_ref[...], w_ref[...],
                (((1,), (1,)), ((), ())),
                preferred_element_type=jnp.float32,
            )
            + b_ref[...]
        )

        if tail:
            @pl.when(i < nv - 1)
            def _copy_full():
                pltpu.make_async_copy(
                    o_vmem.at[s], o_hbm.at[:, pl.ds(i * block_v, block_v)],
                    sems.at[s],
                ).start()

            @pl.when(i == nv - 1)
            def _copy_tail():
                pltpu.make_async_copy(
                    o_vmem.at[s, :, pl.ds(0, tail)],
                    o_hbm.at[:, pl.ds(nv_full * block_v, tail)],
                    sems.at[s],
                ).start()
        else:
            pltpu.make_async_copy(
                o_vmem.at[s], o_hbm.at[:, pl.ds(i * block_v, block_v)],
                sems.at[s],
            ).start()

        @pl.when(i == nv - 1)
        def _drain():
            for k in range(min(ring, nv)):
                step = nv - 1 - k
                slot = step % ring
                if tail and step == nv - 1:
                    pltpu.make_async_copy(
                        o_vmem.at[slot, :, pl.ds(0, tail)],
                        o_hbm.at[:, pl.ds(nv_full * block_v, tail)],
                        sems.at[slot],
                    ).wait()
                else:
                    pltpu.make_async_copy(
                        o_vmem.at[slot],
                        o_hbm.at[:, pl.ds(0, block_v)],
                        sems.at[slot],
                    ).wait()

    return pl.pallas_call(
        mm_k,
        grid=(nv,),
        in_specs=[
            pl.BlockSpec((B, H), lambda i: (0, 0)),
            pl.BlockSpec((block_v, H), lambda i: (i, 0)),
            pl.BlockSpec((1, block_v), lambda i: (0, i)),
        ],
        out_specs=pl.BlockSpec(memory_space=pl.ANY),
        out_shape=jax.ShapeDtypeStruct((B, V), jnp.float32),
        scratch_shapes=[
            pltpu.VMEM((ring, B, block_v), jnp.float32),
            pltpu.SemaphoreType.DMA((ring,)),
        ],
    )(x, W, b.reshape(1, V))


def _project_sliver(out, x, W, b):
    B, H = x.shape
    V = W.shape[0]
    if V % 128 == 0:
        return out
    blk = V // 128

    def sliver_k(o_in_ref, x_ref, w_ref, b_ref, o_ref):
        del o_in_ref
        o_ref[...] = (
            lax.dot_general(
                x_ref[...], w_ref[...],
                (((1,), (1,)), ((), ())),
                preferred_element_type=jnp.float32,
            )
            + b_ref[...]
        )

    return pl.pallas_call(
        sliver_k,
        grid=(1,),
        in_specs=[
            pl.BlockSpec(memory_space=pl.ANY),
            pl.BlockSpec((B, H), lambda i: (0, 0)),
            pl.BlockSpec((128, H), lambda i: (blk, 0)),
            pl.BlockSpec((1, 128), lambda i: (0, blk)),
        ],
        out_specs=pl.BlockSpec((B, 128), lambda i: (0, blk)),
        out_shape=jax.ShapeDtypeStruct((B, V), jnp.float32),
        input_output_aliases={0: 0},
    )(out, x, W, b.reshape(1, V))


def kernel(input_ids, emb_table, W, b):
    x = _gather_rows_sc(input_ids.astype(jnp.int32), emb_table)
    out = _project_main(x, W, b)
    return _project_sliver(out, x, W, b)


# speedup vs baseline: 1.0032x; 1.0032x over previous
import functools

import jax
import jax.numpy as jnp
from jax import lax
from jax.experimental import pallas as pl
from jax.experimental.pallas import tpu as pltpu
from jax.experimental.pallas import tpu_sc as plsc


def _gather_rows_sc(input_ids, emb_table):
    B = input_ids.shape[0]
    V, H = emb_table.shape
    info = plsc.get_sparse_core_info()
    nw = info.num_cores * info.num_subcores
    b_per_w = B // nw

    mesh = plsc.VectorSubcoreMesh(core_axis_name="c", subcore_axis_name="s")

    @functools.partial(
        pl.kernel,
        mesh=mesh,
        out_type=jax.ShapeDtypeStruct((B, H), jnp.float32),
        compiler_params=pltpu.CompilerParams(use_tc_tiling_on_sc=False),
        scratch_types=[
            pltpu.VMEM((b_per_w,), jnp.int32),
            pltpu.VMEM((b_per_w, H), jnp.float32),
            pltpu.SemaphoreType.DMA,
        ],
    )
    def gather_k(idx_hbm, table_hbm, out_hbm, idx_v, rows_v, sem):
        wid = lax.axis_index("s") * info.num_cores + lax.axis_index("c")
        base = wid * b_per_w
        pltpu.sync_copy(idx_hbm.at[pl.ds(base, b_per_w)], idx_v)
        pltpu.async_copy(table_hbm.at[idx_v], rows_v, sem).wait()
        pltpu.sync_copy(rows_v, out_hbm.at[pl.ds(base, b_per_w)])

    return gather_k(input_ids, emb_table)


def _project_tc(x, W, b, block_v=2048, ring=4):
    B, H = x.shape
    V = W.shape[0]
    nv = pl.cdiv(V, block_v)
    last = nv - 1
    tail = V - last * block_v
    tail_al = (tail // 128) * 128
    sliver = tail - tail_al
    assert tail_al + 128 <= block_v

    def mm_k(x_ref, w_ref, b_ref, o_hbm, o_vmem, sems):
        i = pl.program_id(0)
        s = lax.rem(i, ring)

        for t in range(ring):
            @pl.when((s == t) & (i >= ring))
            def _wait_slot(t=t):
                pltpu.make_async_copy(
                    o_vmem.at[t], o_hbm.at[:, pl.ds(0, block_v)], sems.at[t]
                ).wait()

        o_vmem[s, :, :] = (
            lax.dot_general(
                x_ref[...], w_ref[...],
                (((1,), (1,)), ((), ())),
                preferred_element_type=jnp.float32,
            )
            + b_ref[...]
        )

        for t in range(ring):
            @pl.when((s == t) & (i < last))
            def _copy_full(t=t):
                pltpu.make_async_copy(
                    o_vmem.at[t], o_hbm.at[:, pl.ds(i * block_v, block_v)],
                    sems.at[t],
                ).start()

        ls = last % ring

        @pl.when(i == last)
        def _copy_tail():
            if tail_al:
                pltpu.make_async_copy(
                    o_vmem.at[ls, :, pl.ds(0, tail_al)],
                    o_hbm.at[:, pl.ds(last * block_v, tail_al)],
                    sems.at[ls],
                ).start()
            if sliver:
                off = pl.multiple_of(
                    lax.max(i, last * block_v + tail_al), 128
                )
                pltpu.make_async_copy(
                    o_vmem.at[ls, :, pl.ds(tail_al, 128)],
                    o_hbm.at[:, pl.ds(off, 128)],
                    sems.at[ls],
                ).start()

            for k in range(min(ring, nv)):
                step = last - k
                t = step % ring
                if step == last:
                    if tail_al:
                        pltpu.make_async_copy(
                            o_vmem.at[t, :, pl.ds(0, tail_al)],
                            o_hbm.at[:, pl.ds(last * block_v, tail_al)],
                            sems.at[t],
                        ).wait()
                    if sliver:
                        off = pl.multiple_of(
                            lax.max(i, last * block_v + tail_al), 128
                        )
                        pltpu.make_async_copy(
                            o_vmem.at[t, :, pl.ds(tail_al, 128)],
                            o_hbm.at[:, pl.ds(off, 128)],
                            sems.at[t],
                        ).wait()
                else:
                    pltpu.make_async_copy(
                        o_vmem.at[t], o_hbm.at[:, pl.ds(0, block_v)],
                        sems.at[t],
                    ).wait()

    return pl.pallas_call(
        mm_k,
        grid=(nv,),
        in_specs=[
            pl.BlockSpec((B, H), lambda i: (0, 0)),
            pl.BlockSpec((block_v, H), lambda i: (i, 0)),
            pl.BlockSpec((1, block_v), lambda i: (0, i)),
        ],
        out_specs=pl.BlockSpec(memory_space=pl.ANY),
        out_shape=jax.ShapeDtypeStruct((B, V), jnp.float32),
        scratch_shapes=[
            pltpu.VMEM((ring, B, block_v), jnp.float32),
            pltpu.SemaphoreType.DMA((ring,)),
        ],
    )(x, W, b.reshape(1, V))


def kernel(input_ids, emb_table, W, b):
    x = _gather_rows_sc(input_ids.astype(jnp.int32), emb_table)
    return _project_tc(x, W, b)


# speedup vs baseline: 1.9235x; 1.9174x over previous
import functools

import jax
import jax.numpy as jnp
from jax import lax
from jax.experimental import pallas as pl
from jax.experimental.pallas import tpu as pltpu
from jax.experimental.pallas import tpu_sc as plsc


def _gather_rows_sc(input_ids, emb_table):
    B = input_ids.shape[0]
    V, H = emb_table.shape
    info = plsc.get_sparse_core_info()
    nw = info.num_cores * info.num_subcores
    b_per_w = B // nw

    mesh = plsc.VectorSubcoreMesh(core_axis_name="c", subcore_axis_name="s")

    @functools.partial(
        pl.kernel,
        mesh=mesh,
        out_type=jax.ShapeDtypeStruct((B, H), jnp.float32),
        compiler_params=pltpu.CompilerParams(use_tc_tiling_on_sc=False),
        scratch_types=[
            pltpu.VMEM((b_per_w,), jnp.int32),
            pltpu.VMEM((b_per_w, H), jnp.float32),
            pltpu.SemaphoreType.DMA,
        ],
    )
    def gather_k(idx_hbm, table_hbm, out_hbm, idx_v, rows_v, sem):
        wid = lax.axis_index("s") * info.num_cores + lax.axis_index("c")
        base = wid * b_per_w
        pltpu.sync_copy(idx_hbm.at[pl.ds(base, b_per_w)], idx_v)
        pltpu.async_copy(table_hbm.at[idx_v], rows_v, sem).wait()
        pltpu.sync_copy(rows_v, out_hbm.at[pl.ds(base, b_per_w)])

    return gather_k(input_ids, emb_table)


def _project_tc_t(x, W, b, block_v=2048, ring=4):
    B, H = x.shape
    V = W.shape[0]
    nv = pl.cdiv(V, block_v)
    last = nv - 1
    tail = V - last * block_v

    def mm_k(w_ref, x_ref, b_ref, o_hbm, o_vmem, sems):
        i = pl.program_id(0)
        s = lax.rem(i, ring)

        for t in range(ring):
            @pl.when((s == t) & (i >= ring))
            def _wait_slot(t=t):
                pltpu.make_async_copy(
                    o_vmem.at[t], o_hbm.at[pl.ds(0, block_v), :], sems.at[t]
                ).wait()

        o_vmem[s, :, :] = (
            lax.dot_general(
                w_ref[...], x_ref[...],
                (((1,), (1,)), ((), ())),
                preferred_element_type=jnp.float32,
            )
            + b_ref[...]
        )

        for t in range(ring):
            @pl.when((s == t) & (i < last))
            def _copy_full(t=t):
                pltpu.make_async_copy(
                    o_vmem.at[t], o_hbm.at[pl.ds(i * block_v, block_v), :],
                    sems.at[t],
                ).start()

        ls = last % ring

        @pl.when(i == last)
        def _copy_tail_and_drain():
            pltpu.make_async_copy(
                o_vmem.at[ls, pl.ds(0, tail), :],
                o_hbm.at[pl.ds(last * block_v, tail), :],
                sems.at[ls],
            ).start()
            for k in range(min(ring, nv)):
                step = last - k
                t = step % ring
                if step == last:
                    pltpu.make_async_copy(
                        o_vmem.at[t, pl.ds(0, tail), :],
                        o_hbm.at[pl.ds(last * block_v, tail), :],
                        sems.at[t],
                    ).wait()
                else:
                    pltpu.make_async_copy(
                        o_vmem.at[t], o_hbm.at[pl.ds(0, block_v), :],
                        sems.at[t],
                    ).wait()

    return pl.pallas_call(
        mm_k,
        grid=(nv,),
        in_specs=[
            pl.BlockSpec((block_v, H), lambda i: (i, 0)),
            pl.BlockSpec((B, H), lambda i: (0, 0)),
            pl.BlockSpec((block_v, 1), lambda i: (i, 0)),
        ],
        out_specs=pl.BlockSpec(memory_space=pl.ANY),
        out_shape=jax.ShapeDtypeStruct((V, B), jnp.float32),
        scratch_shapes=[
            pltpu.VMEM((ring, block_v, B), jnp.float32),
            pltpu.SemaphoreType.DMA((ring,)),
        ],
    )(W, x, b.reshape(V, 1))


def kernel(input_ids, emb_table, W, b):
    x = _gather_rows_sc(input_ids.astype(jnp.int32), emb_table)
    out_t = _project_tc_t(x, W, b)
    return out_t.T


# speedup vs baseline: 1.9264x; 1.0015x over previous
import functools

import jax
import jax.numpy as jnp
from jax import lax
from jax.experimental import pallas as pl
from jax.experimental.pallas import tpu as pltpu
from jax.experimental.pallas import tpu_sc as plsc


def _gather_rows_sc(input_ids, emb_table):
    B = input_ids.shape[0]
    V, H = emb_table.shape
    info = plsc.get_sparse_core_info()
    nw = info.num_cores * info.num_subcores
    b_per_w = B // nw

    mesh = plsc.VectorSubcoreMesh(core_axis_name="c", subcore_axis_name="s")

    @functools.partial(
        pl.kernel,
        mesh=mesh,
        out_type=jax.ShapeDtypeStruct((B, H), jnp.float32),
        compiler_params=pltpu.CompilerParams(use_tc_tiling_on_sc=False),
        scratch_types=[
            pltpu.VMEM((b_per_w,), jnp.int32),
            pltpu.VMEM((b_per_w, H), jnp.float32),
            pltpu.SemaphoreType.DMA,
        ],
    )
    def gather_k(idx_hbm, table_hbm, out_hbm, idx_v, rows_v, sem):
        wid = lax.axis_index("s") * info.num_cores + lax.axis_index("c")
        base = wid * b_per_w
        pltpu.sync_copy(idx_hbm.at[pl.ds(base, b_per_w)], idx_v)
        pltpu.async_copy(table_hbm.at[idx_v], rows_v, sem).wait()
        pltpu.sync_copy(rows_v, out_hbm.at[pl.ds(base, b_per_w)])

    return gather_k(input_ids, emb_table)


def _project_tc_t(x, W, b, block_v=2048, ring=6):
    B, H = x.shape
    V = W.shape[0]
    nv = pl.cdiv(V, block_v)
    last = nv - 1
    tail = V - last * block_v

    def mm_k(w_ref, x_ref, b_ref, o_hbm, o_vmem, sems):
        i = pl.program_id(0)
        s = lax.rem(i, ring)

        for t in range(ring):
            @pl.when((s == t) & (i >= ring))
            def _wait_slot(t=t):
                pltpu.make_async_copy(
                    o_vmem.at[t], o_hbm.at[pl.ds(0, block_v), :], sems.at[t]
                ).wait()

        o_vmem[s, :, :] = (
            lax.dot_general(
                w_ref[...].astype(jnp.bfloat16),
                x_ref[...].astype(jnp.bfloat16),
                (((1,), (1,)), ((), ())),
                preferred_element_type=jnp.float32,
            )
            + b_ref[...]
        )

        for t in range(ring):
            @pl.when((s == t) & (i < last))
            def _copy_full(t=t):
                pltpu.make_async_copy(
                    o_vmem.at[t], o_hbm.at[pl.ds(i * block_v, block_v), :],
                    sems.at[t],
                ).start()

        ls = last % ring

        @pl.when(i == last)
        def _copy_tail_and_drain():
            pltpu.make_async_copy(
                o_vmem.at[ls, pl.ds(0, tail), :],
                o_hbm.at[pl.ds(last * block_v, tail), :],
                sems.at[ls],
            ).start()
            for k in range(min(ring, nv)):
                step = last - k
                t = step % ring
                if step == last:
                    pltpu.make_async_copy(
                        o_vmem.at[t, pl.ds(0, tail), :],
                        o_hbm.at[pl.ds(last * block_v, tail), :],
                        sems.at[t],
                    ).wait()
                else:
                    pltpu.make_async_copy(
                        o_vmem.at[t], o_hbm.at[pl.ds(0, block_v), :],
                        sems.at[t],
                    ).wait()

    return pl.pallas_call(
        mm_k,
        grid=(nv,),
        in_specs=[
            pl.BlockSpec((block_v, H), lambda i: (i, 0)),
            pl.BlockSpec((B, H), lambda i: (0, 0)),
            pl.BlockSpec((block_v, 1), lambda i: (i, 0)),
        ],
        out_specs=pl.BlockSpec(memory_space=pl.ANY),
        out_shape=jax.ShapeDtypeStruct((V, B), jnp.float32),
        scratch_shapes=[
            pltpu.VMEM((ring, block_v, B), jnp.float32),
            pltpu.SemaphoreType.DMA((ring,)),
        ],
    )(W, x, b.reshape(V, 1))


def kernel(input_ids, emb_table, W, b):
    x = _gather_rows_sc(input_ids.astype(jnp.int32), emb_table)
    out_t = _project_tc_t(x, W, b)
    return out_t.T
